# baseline (speedup 1.0000x reference)
import jax
import jax.numpy as jnp
from jax.experimental import pallas as pl

_N = 1024 * 1024
_COLS = 1024
_ROWS = _N // _COLS
_INV_T = 10.0


def _body(w_ref, z_ref, o_ref):
    x = (w_ref[...] - z_ref[...]) * _INV_T
    o_ref[...] = jax.nn.sigmoid(x)


def kernel(weights, noises):
    w = weights.reshape(_ROWS, _COLS)
    z = noises.reshape(_ROWS, _COLS)
    grid = 8
    blk = _ROWS // grid
    out = pl.pallas_call(
        _body,
        grid=(grid,),
        in_specs=[
            pl.BlockSpec((blk, _COLS), lambda i: (i, 0)),
            pl.BlockSpec((blk, _COLS), lambda i: (i, 0)),
        ],
        out_specs=pl.BlockSpec((blk, _COLS), lambda i: (i, 0)),
        out_shape=jax.ShapeDtypeStruct((_ROWS, _COLS), jnp.float32),
    )(w, z)
    return out.reshape(_N)


# speedup vs baseline: 2.5360x; 2.5360x over previous
import jax
import jax.numpy as jnp
from jax.experimental import pallas as pl

_N = 1024 * 1024
_COLS = 1024
_ROWS = _N // _COLS
_INV_T = 10.0


def _body(w_ref, z_ref, o_ref):
    x = (w_ref[...] - z_ref[...]) * _INV_T
    o_ref[...] = jax.nn.sigmoid(x)


def kernel(weights, noises):
    grid = 8
    blk = _N // grid
    out = pl.pallas_call(
        _body,
        grid=(grid,),
        in_specs=[
            pl.BlockSpec((blk,), lambda i: (i,)),
            pl.BlockSpec((blk,), lambda i: (i,)),
        ],
        out_specs=pl.BlockSpec((blk,), lambda i: (i,)),
        out_shape=jax.ShapeDtypeStruct((_N,), jnp.float32),
    )(weights, noises)
    return out
